# baseline (speedup 1.0000x reference)
import jax
import jax.numpy as jnp
from jax import lax
from jax.experimental import pallas as pl
from jax.experimental.pallas import tpu as pltpu
from jax.experimental.pallas import tpu_sc as plsc

E = 64
D = 1024
F = 512
T = 2048
M = 64
NT = 96
P = NT * M
L = 16
NSUB = 16
TPW = T // NSUB
CPW = TPW // L


def _splat(x):
    return jnp.full((L,), x, jnp.int32)


def _mask_i32(m):
    return jnp.where(m, jnp.int32(1), jnp.int32(0))


def _route_body(idx_hbm, w_hbm, x_hbm,
                xs_hbm, ws_hbm, dest_hbm, te_hbm, act_hbm,
                idx_v, w_v, lrank_v, row_v, cvm_v, dest_v, te_v,
                rows_v, wrow_v, cnt_s, shared_cnt, sem):
    c = lax.axis_index("c")
    s = lax.axis_index("s")
    iota = lax.iota(jnp.int32, L)

    pltpu.sync_copy(idx_hbm, idx_v)
    pltpu.sync_copy(w_hbm, w_v)

    for e in range(E):
        cnt_s[e] = jnp.int32(0)

    base_tok = s * TPW
    for k in range(CPW):
        v = idx_v[pl.ds(base_tok + k * L, L)]
        lr = jnp.zeros((L,), jnp.int32)
        for j in range(L):
            ej = v[j]
            r = cnt_s[ej]
            cnt_s[ej] = r + 1
            lr = jnp.where(iota == j, _splat(r), lr)
        lrank_v[pl.ds(k * L, L)] = lr

    for r in range(4):
        row = jnp.zeros((L,), jnp.int32)
        for j in range(L):
            row = jnp.where(iota == j, _splat(cnt_s[r * L + j]), row)
        row_v[...] = row
        pltpu.sync_copy(row_v, shared_cnt.at[pl.ds((s * 4 + r) * L, L)])
    plsc.subcore_barrier()
    pltpu.sync_copy(shared_cnt, cvm_v)

    rows = [[cvm_v[pl.ds((s2 * 4 + r) * L, L)] for r in range(4)]
            for s2 in range(NSUB)]
    total = [jnp.zeros((L,), jnp.int32) for _ in range(4)]
    base = [jnp.zeros((L,), jnp.int32) for _ in range(4)]
    for s2 in range(NSUB):
        sel = _mask_i32(_splat(s2) < _splat(s))
        for r in range(4):
            total[r] = total[r] + rows[s2][r]
            base[r] = base[r] + rows[s2][r] * sel

    iota96 = [iota + j * L for j in range(NT // L)]
    te = [_splat(-1) for _ in range(NT // L)]
    ntsum = jnp.int32(0)
    laste = jnp.int32(0)
    for e in range(E):
        cnt = total[e // L][e % L]
        nt = (cnt + (M - 1)) // M
        ts = _splat(ntsum)
        te = [tej + _mask_i32(ij >= ts) for tej, ij in zip(te, iota96)]
        cnt_s[e] = ntsum * M + base[e // L][e % L]
        ntsum = ntsum + nt
        laste = jnp.where(nt > 0, jnp.int32(e), laste)

    @pl.when(jnp.logical_and(c == 0, s == 0))
    def _write_te():
        for j in range(NT // L):
            te_v[pl.ds(j * L, L)] = jnp.minimum(te[j], _splat(laste))
        pltpu.sync_copy(te_v, te_hbm)
        for j in range(NT // L):
            te_v[pl.ds(j * L, L)] = _mask_i32(iota96[j] < _splat(ntsum))
        pltpu.sync_copy(te_v, act_hbm)

    if True:
        def chunk_body(k, _):
            kk = 2 * k + c
            tok0 = base_tok + kk * L
            v = idx_v[pl.ds(tok0, L)]
            lr = lrank_v[pl.ds(kk * L, L)]
            dst = jnp.zeros((L,), jnp.int32)
            for j in range(L):
                dj = cnt_s[v[j]] + lr[j]
                dst = jnp.where(iota == j, _splat(dj), dst)
            dest_v[...] = dst
            pltpu.sync_copy(dest_v, dest_hbm.at[pl.ds(tok0, L)])
            pltpu.sync_copy(x_hbm.at[pl.ds(tok0, L)], rows_v)
            pltpu.async_copy(rows_v, xs_hbm.at[dest_v], sem).wait()
            wv = w_v[pl.ds(tok0, L)]
            for j in range(L):
                wrow_v[j, pl.ds(0, L)] = jnp.full((L,), wv[j], jnp.float32)
            pltpu.async_copy(wrow_v, ws_hbm.at[dest_v], sem).wait()
            return 0

        lax.fori_loop(0, CPW // 2, chunk_body, 0)


def _route(idx, w, x):
    kfn = pl.kernel(
        _route_body,
        out_type=(
            jax.ShapeDtypeStruct((P, D), jnp.float32),
            jax.ShapeDtypeStruct((P, 128), jnp.float32),
            jax.ShapeDtypeStruct((T,), jnp.int32),
            jax.ShapeDtypeStruct((NT,), jnp.int32),
            jax.ShapeDtypeStruct((NT,), jnp.int32),
        ),
        mesh=plsc.VectorSubcoreMesh(core_axis_name="c", subcore_axis_name="s"),
        scratch_types=[
            pltpu.VMEM((T,), jnp.int32),
            pltpu.VMEM((T,), jnp.float32),
            pltpu.VMEM((TPW,), jnp.int32),
            pltpu.VMEM((L,), jnp.int32),
            pltpu.VMEM((NSUB * 4 * L,), jnp.int32),
            pltpu.VMEM((L,), jnp.int32),
            pltpu.VMEM((NT,), jnp.int32),
            pltpu.VMEM((L, D), jnp.float32),
            pltpu.VMEM((L, 128), jnp.float32),
            pltpu.SMEM((E,), jnp.int32),
            pltpu.VMEM_SHARED((NSUB * 4 * L,), jnp.int32),
            pltpu.SemaphoreType.DMA,
        ],
    )
    return kfn(idx, w, x)


def _combine_body(ys_hbm, dest_hbm, out_hbm, idx_v, rows_v, sem):
    c = lax.axis_index("c")
    s = lax.axis_index("s")
    wid = s * 2 + c
    per_w = T // 32
    base = wid * per_w

    def body(k, _):
        off = base + k * L
        pltpu.sync_copy(dest_hbm.at[pl.ds(off, L)], idx_v)
        pltpu.async_copy(ys_hbm.at[idx_v], rows_v, sem).wait()
        pltpu.sync_copy(rows_v, out_hbm.at[pl.ds(off, L)])
        return 0

    lax.fori_loop(0, per_w // L, body, 0)


def _combine(ys, dest):
    kfn = pl.kernel(
        _combine_body,
        out_type=jax.ShapeDtypeStruct((T, D), jnp.float32),
        mesh=plsc.VectorSubcoreMesh(core_axis_name="c", subcore_axis_name="s"),
        scratch_types=[
            pltpu.VMEM((L,), jnp.int32),
            pltpu.VMEM((L, D), jnp.float32),
            pltpu.SemaphoreType.DMA,
        ],
    )
    return kfn(ys, dest)


def _mlp_kernel(te_ref, act_ref, x_ref, wg_ref, wu_ref, wd_ref, ws_ref,
                out_ref, acc_ref):
    i = pl.program_id(0)
    j = pl.program_id(1)

    @pl.when(act_ref[i] == 1)
    def _compute():
        @pl.when(j == 0)
        def _gate():
            acc_ref[...] = jnp.dot(x_ref[...], wg_ref[0],
                                   preferred_element_type=jnp.float32)

        @pl.when(j == 1)
        def _up():
            g = acc_ref[...]
            u = jnp.dot(x_ref[...], wu_ref[0],
                        preferred_element_type=jnp.float32)
            acc_ref[...] = (g * jax.nn.sigmoid(g)) * u

        @pl.when(j == 2)
        def _down():
            y = jnp.dot(acc_ref[...], wd_ref[0],
                        preferred_element_type=jnp.float32)
            out_ref[...] = y * ws_ref[:, 0:1]


def _grouped_mlp(tile_expert, active, xs, Wg, Wu, Wd, ws):
    def _delayed(i, j, stage):
        return jnp.where(j >= stage, i, jnp.maximum(i - 1, 0))

    grid_spec = pltpu.PrefetchScalarGridSpec(
        num_scalar_prefetch=2,
        grid=(NT, 3),
        in_specs=[
            pl.BlockSpec((M, D), lambda i, j, te, act: (i, 0)),
            pl.BlockSpec((1, D, F), lambda i, j, te, act: (te[i], 0, 0)),
            pl.BlockSpec((1, D, F),
                         lambda i, j, te, act: (te[_delayed(i, j, 1)], 0, 0)),
            pl.BlockSpec((1, F, D),
                         lambda i, j, te, act: (te[_delayed(i, j, 2)], 0, 0)),
            pl.BlockSpec((M, 128), lambda i, j, te, act: (i, 0)),
        ],
        out_specs=pl.BlockSpec((M, D), lambda i, j, te, act: (i, 0)),
        scratch_shapes=[pltpu.VMEM((M, F), jnp.float32)],
    )
    return pl.pallas_call(
        _mlp_kernel,
        grid_spec=grid_spec,
        out_shape=jax.ShapeDtypeStruct((P, D), jnp.float32),
        compiler_params=pltpu.CompilerParams(
            dimension_semantics=("arbitrary", "arbitrary"),
        ),
    )(tile_expert, active, xs, Wg, Wu, Wd, ws)


def kernel(hidden_states, top_k_index, top_k_weights, Wg, Wu, Wd):
    idx = top_k_index.astype(jnp.int32)[:, 0]
    w = top_k_weights[:, 0]
    xs, ws, dest, tile_expert, active = _route(idx, w, hidden_states)
    ys = _grouped_mlp(tile_expert, active, xs, Wg, Wu, Wd, ws)
    return _combine(ys, dest)


# speedup vs baseline: 1.4268x; 1.4268x over previous
import jax
import jax.numpy as jnp
from jax import lax
from jax.experimental import pallas as pl
from jax.experimental.pallas import tpu as pltpu
from jax.experimental.pallas import tpu_sc as plsc

E = 64
D = 1024
F = 512
T = 2048
M = 64
NT = 96
P = NT * M
L = 16
NSUB = 16
TPW = T // NSUB
CPW = TPW // L


def _splat(x):
    return jnp.full((L,), x, jnp.int32)


def _mask_i32(m):
    return jnp.where(m, jnp.int32(1), jnp.int32(0))


def _route_body(idx_hbm, w_hbm, x_hbm,
                xs_hbm, ws_hbm, dest_hbm, te_hbm, act_hbm,
                idx_v, w_v, lrank_v, row_v, cvm_v, dest_v, te_v,
                rows_v, wrow_v, cnt_s, shared_cnt, sem):
    c = lax.axis_index("c")
    s = lax.axis_index("s")
    iota = lax.iota(jnp.int32, L)

    pltpu.sync_copy(idx_hbm, idx_v)
    pltpu.sync_copy(w_hbm, w_v)

    for e in range(E):
        cnt_s[e] = jnp.int32(0)

    base_tok = s * TPW
    for k in range(CPW):
        v = idx_v[pl.ds(base_tok + k * L, L)]
        lr = jnp.zeros((L,), jnp.int32)
        for j in range(L):
            ej = v[j]
            r = cnt_s[ej]
            cnt_s[ej] = r + 1
            lr = jnp.where(iota == j, _splat(r), lr)
        lrank_v[pl.ds(k * L, L)] = lr

    for r in range(4):
        row = jnp.zeros((L,), jnp.int32)
        for j in range(L):
            row = jnp.where(iota == j, _splat(cnt_s[r * L + j]), row)
        row_v[...] = row
        pltpu.sync_copy(row_v, shared_cnt.at[pl.ds((s * 4 + r) * L, L)])
    plsc.subcore_barrier()
    pltpu.sync_copy(shared_cnt, cvm_v)

    rows = [[cvm_v[pl.ds((s2 * 4 + r) * L, L)] for r in range(4)]
            for s2 in range(NSUB)]
    total = [jnp.zeros((L,), jnp.int32) for _ in range(4)]
    base = [jnp.zeros((L,), jnp.int32) for _ in range(4)]
    for s2 in range(NSUB):
        sel = _mask_i32(_splat(s2) < _splat(s))
        for r in range(4):
            total[r] = total[r] + rows[s2][r]
            base[r] = base[r] + rows[s2][r] * sel

    iota96 = [iota + j * L for j in range(NT // L)]
    te = [_splat(-1) for _ in range(NT // L)]
    ntsum = jnp.int32(0)
    laste = jnp.int32(0)
    for e in range(E):
        cnt = total[e // L][e % L]
        nt = (cnt + (M - 1)) // M
        ts = _splat(ntsum)
        te = [tej + _mask_i32(ij >= ts) for tej, ij in zip(te, iota96)]
        cnt_s[e] = ntsum * M + base[e // L][e % L]
        ntsum = ntsum + nt
        laste = jnp.where(nt > 0, jnp.int32(e), laste)

    @pl.when(jnp.logical_and(c == 0, s == 0))
    def _write_te():
        for j in range(NT // L):
            te_v[pl.ds(j * L, L)] = jnp.minimum(te[j], _splat(laste))
        pltpu.sync_copy(te_v, te_hbm)
        for j in range(NT // L):
            te_v[pl.ds(j * L, L)] = _mask_i32(iota96[j] < _splat(ntsum))
        pltpu.sync_copy(te_v, act_hbm)

    if True:
        def chunk_body(k, _):
            kk = 2 * k + c
            tok0 = base_tok + kk * L
            v = idx_v[pl.ds(tok0, L)]
            lr = lrank_v[pl.ds(kk * L, L)]
            dst = jnp.zeros((L,), jnp.int32)
            for j in range(L):
                dj = cnt_s[v[j]] + lr[j]
                dst = jnp.where(iota == j, _splat(dj), dst)
            dest_v[...] = dst
            pltpu.sync_copy(dest_v, dest_hbm.at[pl.ds(tok0, L)])
            pltpu.sync_copy(x_hbm.at[pl.ds(tok0, L)], rows_v)
            pltpu.async_copy(rows_v, xs_hbm.at[dest_v], sem).wait()
            wv = w_v[pl.ds(tok0, L)]
            for j in range(L):
                wrow_v[j, pl.ds(0, L)] = jnp.full((L,), wv[j], jnp.float32)
            pltpu.async_copy(wrow_v, ws_hbm.at[dest_v], sem).wait()
            return 0

        lax.fori_loop(0, CPW // 2, chunk_body, 0)


def _route(idx, w, x):
    kfn = pl.kernel(
        _route_body,
        out_type=(
            jax.ShapeDtypeStruct((P, D), jnp.float32),
            jax.ShapeDtypeStruct((P, 128), jnp.float32),
            jax.ShapeDtypeStruct((T,), jnp.int32),
            jax.ShapeDtypeStruct((NT,), jnp.int32),
            jax.ShapeDtypeStruct((NT,), jnp.int32),
        ),
        mesh=plsc.VectorSubcoreMesh(core_axis_name="c", subcore_axis_name="s"),
        scratch_types=[
            pltpu.VMEM((T,), jnp.int32),
            pltpu.VMEM((T,), jnp.float32),
            pltpu.VMEM((TPW,), jnp.int32),
            pltpu.VMEM((L,), jnp.int32),
            pltpu.VMEM((NSUB * 4 * L,), jnp.int32),
            pltpu.VMEM((L,), jnp.int32),
            pltpu.VMEM((NT,), jnp.int32),
            pltpu.VMEM((L, D), jnp.float32),
            pltpu.VMEM((L, 128), jnp.float32),
            pltpu.SMEM((E,), jnp.int32),
            pltpu.VMEM_SHARED((NSUB * 4 * L,), jnp.int32),
            pltpu.SemaphoreType.DMA,
        ],
    )
    return kfn(idx, w, x)


def _combine_body(ys_hbm, dest_hbm, out_hbm, idx_v, rows_v, sem):
    c = lax.axis_index("c")
    s = lax.axis_index("s")
    wid = s * 2 + c
    per_w = T // 32
    base = wid * per_w

    def body(k, _):
        off = base + k * L
        pltpu.sync_copy(dest_hbm.at[pl.ds(off, L)], idx_v)
        pltpu.async_copy(ys_hbm.at[idx_v], rows_v, sem).wait()
        pltpu.sync_copy(rows_v, out_hbm.at[pl.ds(off, L)])
        return 0

    lax.fori_loop(0, per_w // L, body, 0)


def _combine(ys, dest):
    kfn = pl.kernel(
        _combine_body,
        out_type=jax.ShapeDtypeStruct((T, D), jnp.float32),
        mesh=plsc.VectorSubcoreMesh(core_axis_name="c", subcore_axis_name="s"),
        scratch_types=[
            pltpu.VMEM((L,), jnp.int32),
            pltpu.VMEM((L, D), jnp.float32),
            pltpu.SemaphoreType.DMA,
        ],
    )
    return kfn(ys, dest)


def _mlp_kernel(te_ref, act_ref, x_ref, wg_ref, wu_ref, wd_ref, ws_ref,
                out_ref):
    i = pl.program_id(0)

    @pl.when(act_ref[i] == 1)
    def _compute():
        x = x_ref[...]
        g = jnp.dot(x, wg_ref[0], preferred_element_type=jnp.float32)
        u = jnp.dot(x, wu_ref[0], preferred_element_type=jnp.float32)
        h = (g * jax.nn.sigmoid(g)) * u
        y = jnp.dot(h, wd_ref[0], preferred_element_type=jnp.float32)
        out_ref[...] = y * ws_ref[:, 0:1]


def _grouped_mlp(tile_expert, active, xs, Wg, Wu, Wd, ws):
    grid_spec = pltpu.PrefetchScalarGridSpec(
        num_scalar_prefetch=2,
        grid=(NT,),
        in_specs=[
            pl.BlockSpec((M, D), lambda i, te, act: (i, 0)),
            pl.BlockSpec((1, D, F), lambda i, te, act: (te[i], 0, 0)),
            pl.BlockSpec((1, D, F), lambda i, te, act: (te[i], 0, 0)),
            pl.BlockSpec((1, F, D), lambda i, te, act: (te[i], 0, 0)),
            pl.BlockSpec((M, 128), lambda i, te, act: (i, 0)),
        ],
        out_specs=pl.BlockSpec((M, D), lambda i, te, act: (i, 0)),
    )
    return pl.pallas_call(
        _mlp_kernel,
        grid_spec=grid_spec,
        out_shape=jax.ShapeDtypeStruct((P, D), jnp.float32),
        compiler_params=pltpu.CompilerParams(
            dimension_semantics=("arbitrary",),
        ),
    )(tile_expert, active, xs, Wg, Wu, Wd, ws)


def kernel(hidden_states, top_k_index, top_k_weights, Wg, Wu, Wd):
    idx = top_k_index.astype(jnp.int32)[:, 0]
    w = top_k_weights[:, 0]
    xs, ws, dest, tile_expert, active = _route(idx, w, hidden_states)
    ys = _grouped_mlp(tile_expert, active, xs, Wg, Wu, Wd, ws)
    return _combine(ys, dest)
